# baseline (speedup 1.0000x reference)
import functools

import jax
import jax.numpy as jnp
import numpy as np
from jax.experimental import pallas as pl

IMG_SIZE = 1024.0
NB = 50
NCH = 5
PCH = 256
PHALF = 128
KP = 8

_SIN_C = tuple(np.float32(v) for v in (
    6.27864302414131, -41.09402466685748, 77.9331002172944,
    -56.09356556625044))
_COS_C = tuple(np.float32(v) for v in (
    0.9986094721939437, -19.5558858697705, 61.14161055490837,
    -59.673518593708955))


def _body(pk_ref, w_ref, tab_ref, out_ref):
    bb = pk_ref.shape[0]
    rows = bb * NB
    dn = (((1,), (0,)), ((), ()))

    um = jax.lax.dot_general(
        pk_ref[...].reshape(rows, KP), w_ref[...],
        dimension_numbers=dn,
        preferred_element_type=jnp.float32)
    u = um[:, :PHALF]
    d = um[:, PHALF:]

    f = u - jnp.round(u)
    z = f * f
    s = _SIN_C[3]
    for c in _SIN_C[2::-1]:
        s = s * z + c
    s = s * f
    c = _COS_C[3]
    for cc in _COS_C[2::-1]:
        c = c * z + cc

    onehot = jnp.maximum(1.0 - jnp.abs(d), 0.0).astype(jnp.bfloat16)
    emb = jax.lax.dot_general(
        onehot, tab_ref[...],
        dimension_numbers=dn,
        preferred_element_type=jnp.float32)
    res = jnp.concatenate([s, c], axis=1) + emb
    out_ref[...] = res.reshape(bb, 1, NB, PCH)


@functools.partial(jax.jit, static_argnames=("block_b",))
def _run(pk, w, tab, block_b=16):
    B = pk.shape[0]
    grid = (B // block_b, NCH)
    out = pl.pallas_call(
        _body,
        grid=grid,
        in_specs=[
            pl.BlockSpec((block_b, 1, NB, KP), lambda i, j: (i, j, 0, 0)),
            pl.BlockSpec((KP, PCH), lambda i, j: (0, 0)),
            pl.BlockSpec((PHALF, PCH), lambda i, j: (0, 0)),
        ],
        out_specs=pl.BlockSpec((block_b, 1, NB, PCH),
                               lambda i, j: (i, j, 0, 0)),
        out_shape=jax.ShapeDtypeStruct((B, NCH, NB, PCH), jnp.float32),
    )(pk, w, tab)
    return out.reshape(B, NCH * NB, PCH)


def kernel(branch_points, mid_points, branch_labels, mid_labels, pe_gauss,
           branch_table, mid_table):
    B = branch_points.shape[0]
    NM = mid_points.shape[1]
    pts = jnp.concatenate(
        [branch_points.reshape(B, 1, NB, 2),
         mid_points.reshape(B, NM // NB, NB, 2)], axis=1)
    xc = pts * jnp.float32(2.0 / IMG_SIZE) - 1.0
    xh = xc.astype(jnp.bfloat16)
    xl = (xc - xh.astype(jnp.float32)).astype(jnp.bfloat16)
    labs = jnp.concatenate(
        [branch_labels.reshape(B, 1, NB),
         mid_labels.reshape(B, NM // NB, NB) + 16],
        axis=1).astype(jnp.bfloat16)[..., None]
    pk = jnp.concatenate(
        [xh, xl, xh, labs, jnp.ones_like(labs)], axis=-1)

    g = pe_gauss.astype(jnp.float32)
    gh = g.astype(jnp.bfloat16)
    gl = (g - gh.astype(jnp.float32)).astype(jnp.bfloat16)
    w = jnp.zeros((KP, PCH), jnp.bfloat16)
    w = w.at[0:2, 0:PHALF].set(gh)
    w = w.at[2:4, 0:PHALF].set(gh)
    w = w.at[4:6, 0:PHALF].set(gl)
    w = w.at[6, PHALF:PCH].set(jnp.bfloat16(1))
    w = w.at[7, PHALF:PCH].set(
        -jnp.arange(PHALF, dtype=jnp.float32).astype(jnp.bfloat16))

    tab = jnp.zeros((PHALF, PCH), jnp.bfloat16)
    tab = tab.at[0:16].set(branch_table.astype(jnp.bfloat16))
    tab = tab.at[16:32].set(mid_table.astype(jnp.bfloat16))

    block_b = 16 if B % 16 == 0 else 8
    return _run(pk, w, tab, block_b=block_b)


# speedup vs baseline: 1.3902x; 1.3902x over previous
import functools

import jax
import jax.numpy as jnp
import numpy as np
from jax.experimental import pallas as pl

IMG_SIZE = 1024.0
NB = 50
NM = 200
NCH = 5
PCH = 256
PHALF = 128

_SIN_C = tuple(np.float32(v) for v in (
    6.27864302414131, -41.09402466685748, 77.9331002172944,
    -56.09356556625044))
_COS_C = tuple(np.float32(v) for v in (
    0.9986094721939437, -19.5558858697705, 61.14161055490837,
    -59.673518593708955))


def _body(bp_ref, mp_ref, bl_ref, ml_ref, g2_ref, c1_ref, ones_ref,
          iota_ref, tab_ref, out_ref):
    bb = bp_ref.shape[0]
    rows = bb * NB
    dn = (((1,), (0,)), ((), ()))
    j = pl.program_id(1)
    is_branch = j == 0

    p = jnp.where(is_branch, bp_ref[...], mp_ref[...])
    u = jax.lax.dot_general(
        p.reshape(rows, 2), g2_ref[...],
        dimension_numbers=dn,
        precision=jax.lax.Precision.HIGHEST,
        preferred_element_type=jnp.float32) + c1_ref[...]
    f = u - jnp.round(u)
    z = f * f
    s = _SIN_C[3]
    for c in _SIN_C[2::-1]:
        s = s * z + c
    s = s * f
    c = _COS_C[3]
    for cc in _COS_C[2::-1]:
        c = c * z + cc

    lab = jnp.where(is_branch, bl_ref[...], ml_ref[...] + 16)
    labb = lab.reshape(rows, 1).astype(jnp.bfloat16)
    labL = jax.lax.dot_general(
        labb, ones_ref[...],
        dimension_numbers=dn,
        preferred_element_type=jnp.float32)
    d = labL - iota_ref[...]
    onehot = jnp.maximum(1.0 - jnp.abs(d), 0.0).astype(jnp.bfloat16)
    emb = jax.lax.dot_general(
        onehot, tab_ref[...],
        dimension_numbers=dn,
        preferred_element_type=jnp.float32)
    res = jnp.concatenate([s, c], axis=1) + emb
    out_ref[...] = res.reshape(bb, 1, NB, PCH)


@functools.partial(jax.jit, static_argnames=("block_b",))
def _run(bp4, mp4, bl3, ml3, g2, c1, ones1, iota1, tab, block_b=16):
    B = bp4.shape[0]
    grid = (B // block_b, NCH)
    out = pl.pallas_call(
        _body,
        grid=grid,
        in_specs=[
            pl.BlockSpec((block_b, 1, NB, 2), lambda i, j: (i, 0, 0, 0)),
            pl.BlockSpec((block_b, 1, NB, 2),
                         lambda i, j: (i, jnp.maximum(j - 1, 0), 0, 0)),
            pl.BlockSpec((block_b, 1, NB, 1), lambda i, j: (i, 0, 0, 0)),
            pl.BlockSpec((block_b, 1, NB, 1),
                         lambda i, j: (i, jnp.maximum(j - 1, 0), 0, 0)),
            pl.BlockSpec((2, PHALF), lambda i, j: (0, 0)),
            pl.BlockSpec((1, PHALF), lambda i, j: (0, 0)),
            pl.BlockSpec((1, PHALF), lambda i, j: (0, 0)),
            pl.BlockSpec((1, PHALF), lambda i, j: (0, 0)),
            pl.BlockSpec((PHALF, PCH), lambda i, j: (0, 0)),
        ],
        out_specs=pl.BlockSpec((block_b, 1, NB, PCH),
                               lambda i, j: (i, j, 0, 0)),
        out_shape=jax.ShapeDtypeStruct((B, NCH, NB, PCH), jnp.float32),
    )(bp4, mp4, bl3, ml3, g2, c1, ones1, iota1, tab)
    return out.reshape(B, NCH * NB, PCH)


def kernel(branch_points, mid_points, branch_labels, mid_labels, pe_gauss,
           branch_table, mid_table):
    B = branch_points.shape[0]
    bp4 = branch_points.reshape(B, 1, NB, 2)
    mp4 = mid_points.reshape(B, NM // NB, NB, 2)
    bl3 = branch_labels.astype(jnp.int32).reshape(B, 1, NB, 1)
    ml3 = mid_labels.astype(jnp.int32).reshape(B, NM // NB, NB, 1)
    g = pe_gauss.astype(jnp.float32)
    g2 = g * jnp.float32(2.0 / IMG_SIZE)
    c1 = -(g[0:1] + g[1:2])
    ones1 = jnp.ones((1, PHALF), jnp.bfloat16)
    iota1 = jnp.arange(PHALF, dtype=jnp.float32)[None, :]
    tab = jnp.zeros((PHALF, PCH), jnp.bfloat16)
    tab = tab.at[0:16].set(branch_table.astype(jnp.bfloat16))
    tab = tab.at[16:32].set(mid_table.astype(jnp.bfloat16))
    block_b = 16 if B % 16 == 0 else 8
    return _run(bp4, mp4, bl3, ml3, g2, c1, ones1, iota1, tab,
                block_b=block_b)


# speedup vs baseline: 1.5246x; 1.0967x over previous
import functools

import jax
import jax.numpy as jnp
import numpy as np
from jax.experimental import pallas as pl

IMG_SIZE = 1024.0
NB = 50
NM = 200
NCH = 5
PCH = 256
PHALF = 128

_SIN_C = tuple(np.float32(v) for v in (
    6.27864302414131, -41.09402466685748, 77.9331002172944,
    -56.09356556625044))
_COS_C = tuple(np.float32(v) for v in (
    0.9986094721939437, -19.5558858697705, 61.14161055490837,
    -59.673518593708955))


def _body(bp_ref, mp_ref, bl_ref, ml_ref, g2_ref, c1_ref, ones_ref,
          iota_ref, tab_ref, out_ref):
    bb = bp_ref.shape[0]
    rows = bb * NB
    dn = (((1,), (0,)), ((), ()))
    j = pl.program_id(1)
    is_branch = j == 0

    p = jnp.where(is_branch, bp_ref[...], mp_ref[...])
    u = jax.lax.dot_general(
        p.reshape(rows, 2), g2_ref[...],
        dimension_numbers=dn,
        precision=jax.lax.Precision.HIGHEST,
        preferred_element_type=jnp.float32) + c1_ref[...]
    f = u - jnp.round(u)
    z = f * f
    s = _SIN_C[3]
    for c in _SIN_C[2::-1]:
        s = s * z + c
    s = s * f
    c = _COS_C[3]
    for cc in _COS_C[2::-1]:
        c = c * z + cc

    lab = jnp.where(is_branch, bl_ref[...], ml_ref[...] + 16)
    labb = lab.reshape(rows, 1).astype(jnp.bfloat16)
    labL = jax.lax.dot_general(
        labb, ones_ref[...],
        dimension_numbers=dn,
        preferred_element_type=jnp.float32)
    d = labL - iota_ref[...]
    onehot = jnp.maximum(1.0 - jnp.abs(d), 0.0).astype(jnp.bfloat16)
    emb = jax.lax.dot_general(
        onehot, tab_ref[...],
        dimension_numbers=dn,
        preferred_element_type=jnp.float32)
    res = jnp.concatenate([s, c], axis=1) + emb
    out_ref[...] = res.reshape(bb, 1, NB, PCH)


@functools.partial(jax.jit, static_argnames=("block_b",))
def _run(bp4, mp4, bl3, ml3, g2, c1, ones1, iota1, tab, block_b=16):
    B = bp4.shape[0]
    grid = (B // block_b, NCH)
    out = pl.pallas_call(
        _body,
        grid=grid,
        in_specs=[
            pl.BlockSpec((block_b, 1, NB, 2), lambda i, j: (i, 0, 0, 0)),
            pl.BlockSpec((block_b, 1, NB, 2),
                         lambda i, j: (i, jnp.maximum(j - 1, 0), 0, 0)),
            pl.BlockSpec((block_b, 1, NB, 1), lambda i, j: (i, 0, 0, 0)),
            pl.BlockSpec((block_b, 1, NB, 1),
                         lambda i, j: (i, jnp.maximum(j - 1, 0), 0, 0)),
            pl.BlockSpec((2, PHALF), lambda i, j: (0, 0)),
            pl.BlockSpec((1, PHALF), lambda i, j: (0, 0)),
            pl.BlockSpec((1, PHALF), lambda i, j: (0, 0)),
            pl.BlockSpec((1, PHALF), lambda i, j: (0, 0)),
            pl.BlockSpec((PHALF, PCH), lambda i, j: (0, 0)),
        ],
        out_specs=pl.BlockSpec((block_b, 1, NB, PCH),
                               lambda i, j: (i, j, 0, 0)),
        out_shape=jax.ShapeDtypeStruct((B, NCH, NB, PCH), jnp.float32),
    )(bp4, mp4, bl3, ml3, g2, c1, ones1, iota1, tab)
    return out.reshape(B, NCH * NB, PCH)


def kernel(branch_points, mid_points, branch_labels, mid_labels, pe_gauss,
           branch_table, mid_table):
    B = branch_points.shape[0]
    bp4 = branch_points.reshape(B, 1, NB, 2)
    mp4 = mid_points.reshape(B, NM // NB, NB, 2)
    bl3 = branch_labels.astype(jnp.int32).reshape(B, 1, NB, 1)
    ml3 = mid_labels.astype(jnp.int32).reshape(B, NM // NB, NB, 1)
    g = pe_gauss.astype(jnp.float32)
    g2 = g * jnp.float32(2.0 / IMG_SIZE)
    c1 = -(g[0:1] + g[1:2])
    ones1 = jnp.ones((1, PHALF), jnp.bfloat16)
    iota1 = jnp.arange(PHALF, dtype=jnp.float32)[None, :]
    tab = jnp.zeros((PHALF, PCH), jnp.bfloat16)
    tab = tab.at[0:16].set(branch_table.astype(jnp.bfloat16))
    tab = tab.at[16:32].set(mid_table.astype(jnp.bfloat16))
    block_b = 32 if B % 32 == 0 else 8
    return _run(bp4, mp4, bl3, ml3, g2, c1, ones1, iota1, tab,
                block_b=block_b)


# speedup vs baseline: 1.5480x; 1.0153x over previous
import functools

import jax
import jax.numpy as jnp
import numpy as np
from jax.experimental import pallas as pl

IMG_SIZE = 1024.0
NB = 50
NM = 200
NCH = 5
PCH = 256
PHALF = 128

_SIN_C = tuple(np.float32(v) for v in (
    6.27864302414131, -41.09402466685748, 77.9331002172944,
    -56.09356556625044))
_COS_C = tuple(np.float32(v) for v in (
    0.9986094721939437, -19.5558858697705, 61.14161055490837,
    -59.673518593708955))


def _body(bp_ref, mp_ref, bl_ref, ml_ref, g2_ref, c1_ref, ones_ref,
          iota_ref, tab_ref, out_ref):
    bb = bp_ref.shape[0]
    rows = bb * NB
    dn = (((1,), (0,)), ((), ()))
    j = pl.program_id(1)
    is_branch = j == 0

    p = jnp.where(is_branch, bp_ref[...], mp_ref[...])
    u = jax.lax.dot_general(
        p.reshape(rows, 2), g2_ref[...],
        dimension_numbers=dn,
        precision=jax.lax.Precision.HIGHEST,
        preferred_element_type=jnp.float32) + c1_ref[...]
    f = u - jnp.round(u)
    z = f * f
    s = _SIN_C[3]
    for c in _SIN_C[2::-1]:
        s = s * z + c
    s = s * f
    c = _COS_C[3]
    for cc in _COS_C[2::-1]:
        c = c * z + cc

    lab = jnp.where(is_branch, bl_ref[...], ml_ref[...] + 16)
    labb = lab.reshape(rows, 1).astype(jnp.bfloat16)
    labL = jax.lax.dot_general(
        labb, ones_ref[...],
        dimension_numbers=dn,
        preferred_element_type=jnp.float32)
    d = labL - iota_ref[...]
    onehot = jnp.maximum(1.0 - jnp.abs(d), 0.0).astype(jnp.bfloat16)
    emb = jax.lax.dot_general(
        onehot, tab_ref[...],
        dimension_numbers=dn,
        preferred_element_type=jnp.float32)
    res = jnp.concatenate([s, c], axis=1) + emb
    out_ref[...] = res.reshape(bb, 1, NB, PCH)


@functools.partial(jax.jit, static_argnames=("block_b",))
def _run(bp4, mp4, bl3, ml3, g2, c1, ones1, iota1, tab, block_b=16):
    B = bp4.shape[0]
    grid = (B // block_b, NCH)
    out = pl.pallas_call(
        _body,
        grid=grid,
        in_specs=[
            pl.BlockSpec((block_b, 1, NB, 2), lambda i, j: (i, 0, 0, 0)),
            pl.BlockSpec((block_b, 1, NB, 2),
                         lambda i, j: (i, jnp.maximum(j - 1, 0), 0, 0)),
            pl.BlockSpec((block_b, 1, NB, 1), lambda i, j: (i, 0, 0, 0)),
            pl.BlockSpec((block_b, 1, NB, 1),
                         lambda i, j: (i, jnp.maximum(j - 1, 0), 0, 0)),
            pl.BlockSpec((2, PHALF), lambda i, j: (0, 0)),
            pl.BlockSpec((1, PHALF), lambda i, j: (0, 0)),
            pl.BlockSpec((1, PHALF), lambda i, j: (0, 0)),
            pl.BlockSpec((1, PHALF), lambda i, j: (0, 0)),
            pl.BlockSpec((PHALF, PCH), lambda i, j: (0, 0)),
        ],
        out_specs=pl.BlockSpec((block_b, 1, NB, PCH),
                               lambda i, j: (i, j, 0, 0)),
        out_shape=jax.ShapeDtypeStruct((B, NCH, NB, PCH), jnp.float32),
    )(bp4, mp4, bl3, ml3, g2, c1, ones1, iota1, tab)
    return out.reshape(B, NCH * NB, PCH)


def kernel(branch_points, mid_points, branch_labels, mid_labels, pe_gauss,
           branch_table, mid_table):
    B = branch_points.shape[0]
    bp4 = branch_points.reshape(B, 1, NB, 2)
    mp4 = mid_points.reshape(B, NM // NB, NB, 2)
    bl3 = branch_labels.astype(jnp.int32).reshape(B, 1, NB, 1)
    ml3 = mid_labels.astype(jnp.int32).reshape(B, NM // NB, NB, 1)
    g = pe_gauss.astype(jnp.float32)
    g2 = g * jnp.float32(2.0 / IMG_SIZE)
    c1 = -(g[0:1] + g[1:2])
    ones1 = jnp.ones((1, PHALF), jnp.bfloat16)
    iota1 = jnp.arange(PHALF, dtype=jnp.float32)[None, :]
    tab = jnp.zeros((PHALF, PCH), jnp.bfloat16)
    tab = tab.at[0:16].set(branch_table.astype(jnp.bfloat16))
    tab = tab.at[16:32].set(mid_table.astype(jnp.bfloat16))
    block_b = 64 if B % 64 == 0 else 8
    return _run(bp4, mp4, bl3, ml3, g2, c1, ones1, iota1, tab,
                block_b=block_b)


# speedup vs baseline: 1.5488x; 1.0005x over previous
import functools

import jax
import jax.numpy as jnp
import numpy as np
from jax.experimental import pallas as pl

IMG_SIZE = 1024.0
NB = 50
NM = 200
NCH = 5
PCH = 256
PHALF = 128

_SIN_C = tuple(np.float32(v) for v in (
    6.27864302414131, -41.09402466685748, 77.9331002172944,
    -56.09356556625044))
_COS_C = tuple(np.float32(v) for v in (
    0.9986094721939437, -19.5558858697705, 61.14161055490837,
    -59.673518593708955))


def _body(bp_ref, mp_ref, bl_ref, ml_ref, g2_ref, c1_ref, ones_ref,
          iota_ref, tab_ref, out_ref):
    bb = bp_ref.shape[0]
    rows = bb * NB
    dn = (((1,), (0,)), ((), ()))
    j = pl.program_id(1)
    is_branch = j == 0

    p = jnp.where(is_branch, bp_ref[...], mp_ref[...])
    u = jax.lax.dot_general(
        p.reshape(rows, 2), g2_ref[...],
        dimension_numbers=dn,
        precision=jax.lax.Precision.HIGHEST,
        preferred_element_type=jnp.float32) + c1_ref[...]
    f = u - jnp.round(u)
    z = f * f
    s = _SIN_C[3]
    for c in _SIN_C[2::-1]:
        s = s * z + c
    s = s * f
    c = _COS_C[3]
    for cc in _COS_C[2::-1]:
        c = c * z + cc

    lab = jnp.where(is_branch, bl_ref[...], ml_ref[...] + 16)
    labb = lab.reshape(rows, 1).astype(jnp.bfloat16)
    labL = jax.lax.dot_general(
        labb, ones_ref[...],
        dimension_numbers=dn,
        preferred_element_type=jnp.float32)
    d = labL - iota_ref[...]
    onehot = jnp.maximum(1.0 - jnp.abs(d), 0.0).astype(jnp.bfloat16)
    emb = jax.lax.dot_general(
        onehot, tab_ref[...],
        dimension_numbers=dn,
        preferred_element_type=jnp.float32)
    res = jnp.concatenate([s, c], axis=1) + emb
    out_ref[...] = res.reshape(bb, 1, NB, PCH)


@functools.partial(jax.jit, static_argnames=("block_b",))
def _run(bp4, mp4, bl3, ml3, g2, c1, ones1, iota1, tab, block_b=16):
    B = bp4.shape[0]
    grid = (B // block_b, NCH)
    out = pl.pallas_call(
        _body,
        grid=grid,
        in_specs=[
            pl.BlockSpec((block_b, 1, NB, 2), lambda i, j: (i, 0, 0, 0)),
            pl.BlockSpec((block_b, 1, NB, 2),
                         lambda i, j: (i, jnp.maximum(j - 1, 0), 0, 0)),
            pl.BlockSpec((block_b, 1, NB, 1), lambda i, j: (i, 0, 0, 0)),
            pl.BlockSpec((block_b, 1, NB, 1),
                         lambda i, j: (i, jnp.maximum(j - 1, 0), 0, 0)),
            pl.BlockSpec((2, PHALF), lambda i, j: (0, 0)),
            pl.BlockSpec((1, PHALF), lambda i, j: (0, 0)),
            pl.BlockSpec((1, PHALF), lambda i, j: (0, 0)),
            pl.BlockSpec((1, PHALF), lambda i, j: (0, 0)),
            pl.BlockSpec((PHALF, PCH), lambda i, j: (0, 0)),
        ],
        out_specs=pl.BlockSpec((block_b, 1, NB, PCH),
                               lambda i, j: (i, j, 0, 0)),
        out_shape=jax.ShapeDtypeStruct((B, NCH, NB, PCH), jnp.float32),
    )(bp4, mp4, bl3, ml3, g2, c1, ones1, iota1, tab)
    return out.reshape(B, NCH * NB, PCH)


def kernel(branch_points, mid_points, branch_labels, mid_labels, pe_gauss,
           branch_table, mid_table):
    B = branch_points.shape[0]
    bp4 = branch_points.reshape(B, 1, NB, 2)
    mp4 = mid_points.reshape(B, NM // NB, NB, 2)
    bl3 = branch_labels.astype(jnp.int32).reshape(B, 1, NB, 1)
    ml3 = mid_labels.astype(jnp.int32).reshape(B, NM // NB, NB, 1)
    g = pe_gauss.astype(jnp.float32)
    g2 = g * jnp.float32(2.0 / IMG_SIZE)
    c1 = -(g[0:1] + g[1:2])
    ones1 = jnp.ones((1, PHALF), jnp.bfloat16)
    iota1 = jnp.arange(PHALF, dtype=jnp.float32)[None, :]
    tab = jnp.zeros((PHALF, PCH), jnp.bfloat16)
    tab = tab.at[0:16].set(branch_table.astype(jnp.bfloat16))
    tab = tab.at[16:32].set(mid_table.astype(jnp.bfloat16))
    block_b = 128 if B % 128 == 0 else 8
    return _run(bp4, mp4, bl3, ml3, g2, c1, ones1, iota1, tab,
                block_b=block_b)


# speedup vs baseline: 1.5863x; 1.0242x over previous
import functools

import jax
import jax.numpy as jnp
import numpy as np
from jax.experimental import pallas as pl

IMG_SIZE = 1024.0
NB = 50
NM = 200
NCH = 5
PCH = 256
PHALF = 128

_SIN_C = tuple(np.float32(v) for v in (
    6.27864302414131, -41.09402466685748, 77.9331002172944,
    -56.09356556625044))
_COS_C = tuple(np.float32(v) for v in (
    0.9986094721939437, -19.5558858697705, 61.14161055490837,
    -59.673518593708955))


def _body(bp_ref, mp_ref, bl_ref, ml_ref, g2_ref, c1_ref, ones_ref,
          iota_ref, tab_ref, out_ref):
    bb = bp_ref.shape[0]
    rows = bb * NB
    dn = (((1,), (0,)), ((), ()))
    j = pl.program_id(1)
    is_branch = j == 0

    p = jnp.where(is_branch, bp_ref[...], mp_ref[...])
    u = jax.lax.dot_general(
        p.reshape(rows, 2), g2_ref[...],
        dimension_numbers=dn,
        precision=jax.lax.Precision.HIGHEST,
        preferred_element_type=jnp.float32) + c1_ref[...]
    f = u - jnp.round(u)
    z = f * f
    s = _SIN_C[3]
    for c in _SIN_C[2::-1]:
        s = s * z + c
    s = s * f
    c = _COS_C[3]
    for cc in _COS_C[2::-1]:
        c = c * z + cc

    lab = jnp.where(is_branch, bl_ref[...], ml_ref[...])
    labL = jax.lax.dot_general(
        lab.reshape(rows, 1), ones_ref[...],
        dimension_numbers=dn,
        preferred_element_type=jnp.float32)
    d = labL - iota_ref[...]
    onehot = jnp.maximum(1.0 - jnp.abs(d), 0.0).astype(jnp.bfloat16)
    emb = jax.lax.dot_general(
        onehot, tab_ref[...],
        dimension_numbers=dn,
        preferred_element_type=jnp.float32)
    res = jnp.concatenate([s, c], axis=1) + emb
    out_ref[...] = res.reshape(bb, 1, NB, PCH)


@functools.partial(jax.jit, static_argnames=("block_b",))
def _run(bp4, mp4, bl3, ml3, g2, c1, ones1, iota1, tab, block_b=16):
    B = bp4.shape[0]
    grid = (B // block_b, NCH)
    out = pl.pallas_call(
        _body,
        grid=grid,
        in_specs=[
            pl.BlockSpec((block_b, 1, NB, 2), lambda i, j: (i, 0, 0, 0)),
            pl.BlockSpec((block_b, 1, NB, 2),
                         lambda i, j: (i, jnp.maximum(j - 1, 0), 0, 0)),
            pl.BlockSpec((block_b, 1, NB, 1), lambda i, j: (i, 0, 0, 0)),
            pl.BlockSpec((block_b, 1, NB, 1),
                         lambda i, j: (i, jnp.maximum(j - 1, 0), 0, 0)),
            pl.BlockSpec((2, PHALF), lambda i, j: (0, 0)),
            pl.BlockSpec((1, PHALF), lambda i, j: (0, 0)),
            pl.BlockSpec((1, PHALF), lambda i, j: (0, 0)),
            pl.BlockSpec((1, PHALF), lambda i, j: (0, 0)),
            pl.BlockSpec((PHALF, PCH), lambda i, j: (0, 0)),
        ],
        out_specs=pl.BlockSpec((block_b, 1, NB, PCH),
                               lambda i, j: (i, j, 0, 0)),
        out_shape=jax.ShapeDtypeStruct((B, NCH, NB, PCH), jnp.float32),
    )(bp4, mp4, bl3, ml3, g2, c1, ones1, iota1, tab)
    return out.reshape(B, NCH * NB, PCH)


def kernel(branch_points, mid_points, branch_labels, mid_labels, pe_gauss,
           branch_table, mid_table):
    B = branch_points.shape[0]
    bp4 = branch_points.reshape(B, 1, NB, 2)
    mp4 = mid_points.reshape(B, NM // NB, NB, 2)
    bl3 = branch_labels.astype(jnp.float32).reshape(B, 1, NB, 1)
    ml3 = (mid_labels + 16).astype(jnp.float32).reshape(B, NM // NB, NB, 1)
    g = pe_gauss.astype(jnp.float32)
    g2 = g * jnp.float32(2.0 / IMG_SIZE)
    c1 = -(g[0:1] + g[1:2])
    ones1 = jnp.ones((1, PHALF), jnp.float32)
    iota1 = jnp.arange(PHALF, dtype=jnp.float32)[None, :]
    tab = jnp.zeros((PHALF, PCH), jnp.bfloat16)
    tab = tab.at[0:16].set(branch_table.astype(jnp.bfloat16))
    tab = tab.at[16:32].set(mid_table.astype(jnp.bfloat16))
    block_b = 64 if B % 64 == 0 else 8
    return _run(bp4, mp4, bl3, ml3, g2, c1, ones1, iota1, tab,
                block_b=block_b)
